# Initial kernel scaffold; baseline (speedup 1.0000x reference)
import jax
import jax.numpy as jnp
from jax.experimental import pallas as pl


def kernel(x, edge_index, edge_weight, W0, b0, W1, b1, W2, b2, Wm1, bm1, Wm2, bm2):
    raise NotImplementedError("write your pallas kernel here")


# speedup vs baseline: 4.9127x; 4.9127x over previous
import functools

import jax
import jax.numpy as jnp
from jax import lax
from jax.experimental import pallas as pl
from jax.experimental.pallas import tpu as pltpu
from jax.experimental.pallas import tpu_sc as plsc

_N = 10000
_E = 320000
_D = 128
_HID = 256
_LBL = 10

_NC, _NS = 2, 16
_NW = _NC * _NS
_C = 80
_EPW = _E // _NW
_CHUNKS = _EPW // _C
_RB = 200
_NB = _N // _RB
_BPT = -(-_NB // _NS)

_BR = 1000


def _make_agg():
    mesh = plsc.VectorSubcoreMesh(
        core_axis_name="c", subcore_axis_name="s",
        num_cores=_NC, num_subcores=_NS)

    @functools.partial(
        pl.kernel,
        out_type=jax.ShapeDtypeStruct((_NC, _N, _D), jnp.float32),
        mesh=mesh,
        scratch_types=[
            pltpu.VMEM((_C,), jnp.int32),
            pltpu.VMEM((_C,), jnp.int32),
            pltpu.VMEM((_C, _D), jnp.float32),
            pltpu.VMEM((_RB, _D), jnp.float32),
            pltpu.VMEM_SHARED((_N, _D), jnp.float32),
            pltpu.SemaphoreType.DMA,
        ],
    )
    def agg(h_hbm, src_hbm, dst_hbm, zero_hbm, out_hbm,
            src_v, dst_v, rows_v, stage_v, acc_s, sem):
        cid = lax.axis_index("c")
        sid = lax.axis_index("s")
        wid = sid * _NC + cid

        @pl.when(cid != 0)
        def _():
            pltpu.sync_copy(zero_hbm, stage_v)

        def ib(i, carry):
            b = sid + i * _NS

            @pl.when(b < _NB)
            def _():
                r = b * _RB

                @pl.when(cid == 0)
                def _():
                    pltpu.sync_copy(h_hbm.at[pl.ds(r, _RB)], stage_v)
                pltpu.sync_copy(stage_v, acc_s.at[pl.ds(r, _RB)])
            return carry
        lax.fori_loop(0, _BPT, ib, 0)

        plsc.subcore_barrier()

        e0 = wid * _EPW

        def eb(j, carry):
            base = e0 + j * _C
            pltpu.sync_copy(src_hbm.at[pl.ds(base, _C)], src_v)
            pltpu.sync_copy(dst_hbm.at[pl.ds(base, _C)], dst_v)
            pltpu.async_copy(h_hbm.at[src_v], rows_v, sem).wait()
            pltpu.sync_copy(rows_v, acc_s.at[dst_v], add=True)
            return carry
        lax.fori_loop(0, _CHUNKS, eb, 0)

        plsc.subcore_barrier()

        def wb(i, carry):
            b = sid + i * _NS

            @pl.when(b < _NB)
            def _():
                r = b * _RB
                pltpu.sync_copy(acc_s.at[pl.ds(r, _RB)], stage_v)
                pltpu.sync_copy(stage_v, out_hbm.at[cid, pl.ds(r, _RB)])
            return carry
        lax.fori_loop(0, _BPT, wb, 0)

    return agg


_agg = _make_agg()


def _dense(c, W, b):
    def body(c_ref, W_ref, b_ref, o_ref):
        comb = c_ref[0] + c_ref[1]
        o_ref[...] = jnp.maximum(comb @ W_ref[...] + b_ref[...], 0.0)

    return pl.pallas_call(
        body,
        grid=(_N // _BR,),
        in_specs=[
            pl.BlockSpec((2, _BR, _D), lambda i: (0, i, 0)),
            pl.BlockSpec((_D, _D), lambda i: (0, 0)),
            pl.BlockSpec((1, _D), lambda i: (0, 0)),
        ],
        out_specs=pl.BlockSpec((_BR, _D), lambda i: (i, 0)),
        out_shape=jax.ShapeDtypeStruct((_N, _D), jnp.float32),
    )(c, W, b.reshape(1, _D))


def _final(c, W2, b2, Wm1, bm1, Wm2, bm2):
    def body(c_ref, W2_ref, b2_ref, Wm1_ref, bm1_ref, Wm2_ref, bm2_ref,
             o_ref):
        h3 = jnp.maximum((c_ref[0] + c_ref[1]) @ W2_ref[...] + b2_ref[...],
                         0.0)
        t = jnp.maximum(h3 @ Wm1_ref[...] + bm1_ref[...], 0.0)
        o_ref[...] = t @ Wm2_ref[...] + bm2_ref[...]

    return pl.pallas_call(
        body,
        grid=(_N // _BR,),
        in_specs=[
            pl.BlockSpec((2, _BR, _D), lambda i: (0, i, 0)),
            pl.BlockSpec((_D, _D), lambda i: (0, 0)),
            pl.BlockSpec((1, _D), lambda i: (0, 0)),
            pl.BlockSpec((_D, _HID), lambda i: (0, 0)),
            pl.BlockSpec((1, _HID), lambda i: (0, 0)),
            pl.BlockSpec((_HID, _LBL), lambda i: (0, 0)),
            pl.BlockSpec((1, _LBL), lambda i: (0, 0)),
        ],
        out_specs=pl.BlockSpec((_BR, _LBL), lambda i: (i, 0)),
        out_shape=jax.ShapeDtypeStruct((_N, _LBL), jnp.float32),
    )(c, W2, b2.reshape(1, _D), Wm1, bm1.reshape(1, _HID),
      Wm2, bm2.reshape(1, _LBL))


def kernel(x, edge_index, edge_weight, W0, b0, W1, b1, W2, b2,
           Wm1, bm1, Wm2, bm2):
    del edge_weight
    src = edge_index[0]
    dst = edge_index[1]
    zeros = jnp.zeros((_RB, _D), jnp.float32)

    c1 = _agg(x, src, dst, zeros)
    h1 = _dense(c1, W0, b0)
    c2 = _agg(h1, src, dst, zeros)
    h2 = _dense(c2, W1, b1)
    c3 = _agg(h2, src, dst, zeros)
    return _final(c3, W2, b2, Wm1, bm1, Wm2, bm2)
